# baseline (speedup 1.0000x reference)
import functools

import jax
import jax.numpy as jnp
from jax import lax
from jax.experimental import pallas as pl
from jax.experimental.pallas import tpu as pltpu
from jax.experimental.pallas import tpu_sc as plsc

B, C, H, W = 8, 64, 32, 32
TOK_PER_B = H * W
N_CODES = 8192
CODE_TILE = 8192
N_CT = N_CODES // CODE_TILE
VR_PER_TILE = CODE_TILE // 8

SC_NC, SC_NS = 2, 16
SC_NW = SC_NC * SC_NS
N_TOK = B * TOK_PER_B
TOK_PER_W = N_TOK // SC_NW


SUB = 8
SUB_ROWS = CODE_TILE // SUB


def _subtile_argmax(s):
    t = s.shape[-1]
    nvr = SUB_ROWS // 8
    v3 = s.reshape(nvr // 2, 16, t)
    a, b = v3[:, :8, :], v3[:, 8:, :]
    cmp = a >= b
    val = jnp.where(cmp, a, b)
    idx = jnp.where(cmp, jnp.int32(0), jnp.int32(1))
    groups = nvr // 2
    k = 1
    while groups > 1:
        groups //= 2
        val = val.reshape(groups, 16, t)
        idx = idx.reshape(groups, 16, t)
        a, b = val[:, :8, :], val[:, 8:, :]
        ia, ib = idx[:, :8, :], idx[:, 8:, :]
        cmp = a >= b
        val = jnp.where(cmp, a, b)
        idx = jnp.where(cmp, ia, ib + jnp.int32(1 << k))
        k += 1
    return val.reshape(8, t), idx.reshape(8, t)


def _argmax_body(x_ref, cb_ref, out_ref):
    xb = x_ref[0]
    norm = jnp.sqrt(jnp.sum(xb * xb, axis=0, keepdims=True))
    xn = xb / jnp.maximum(norm, 1e-12)

    def sub_dot(i):
        return lax.dot_general(
            cb_ref[pl.ds(i * SUB_ROWS, SUB_ROWS), :], xn,
            (((1,), (0,)), ((), ())),
            preferred_element_type=jnp.float32,
            precision=lax.Precision.DEFAULT,
        )

    results = []
    s_prev = sub_dot(0)
    for i in range(1, SUB + 1):
        s_next = sub_dot(i) if i < SUB else None
        v, ix = _subtile_argmax(s_prev)
        results.append((v, ix * 8 + (i - 1) * SUB_ROWS))
        s_prev = s_next

    val, row = results[0]
    srow = lax.broadcasted_iota(jnp.int32, (8, TOK_PER_B), 0)
    row = row + srow
    for v, r in results[1:]:
        r = r + srow
        cmp = val >= v
        val = jnp.where(cmp, val, v)
        row = jnp.where(cmp, row, r)
    sub = 8
    while sub > 1:
        sub //= 2
        a, b = val[:sub], val[sub:]
        ra, rb = row[:sub], row[sub:]
        win_a = (a > b) | ((a == b) & (ra < rb))
        val = jnp.where(win_a, a, b)
        row = jnp.where(win_a, ra, rb)
    out_ref[...] = row.reshape(1, 1, TOK_PER_B)


def _nearest_code(xr, codebook):
    return pl.pallas_call(
        _argmax_body,
        grid=(B,),
        in_specs=[
            pl.BlockSpec((1, C, TOK_PER_B), lambda b: (b, 0, 0)),
            pl.BlockSpec((CODE_TILE, C), lambda b: (0, 0)),
        ],
        out_specs=pl.BlockSpec((1, 1, TOK_PER_B), lambda b: (b, 0, 0)),
        out_shape=jax.ShapeDtypeStruct((B, 1, TOK_PER_B), jnp.int32),
    )(xr, codebook)


@functools.cache
def _make_sc_gather():
    @functools.partial(
        pl.kernel,
        out_type=jax.ShapeDtypeStruct((N_TOK, C), jnp.float32),
        mesh=plsc.VectorSubcoreMesh(core_axis_name="c", subcore_axis_name="s"),
        compiler_params=pltpu.CompilerParams(use_tc_tiling_on_sc=False),
        scratch_types=[
            pltpu.VMEM((TOK_PER_W,), jnp.int32),
            pltpu.VMEM((TOK_PER_W, C), jnp.float32),
            pltpu.SemaphoreType.DMA,
        ],
    )
    def _sc_gather(table_hbm, idx_hbm, out_hbm, idx_v, rows_v, sem):
        wid = lax.axis_index("s") * SC_NC + lax.axis_index("c")
        base = wid * TOK_PER_W
        pltpu.sync_copy(idx_hbm.at[pl.ds(base, TOK_PER_W)], idx_v)
        pltpu.async_copy(table_hbm.at[idx_v], rows_v, sem).wait()
        pltpu.sync_copy(rows_v, out_hbm.at[pl.ds(base, TOK_PER_W)])

    return _sc_gather


def kernel(x, codebook):
    xr = x.reshape(B, C, TOK_PER_B)
    idx = _nearest_code(xr, codebook)
    rows = _make_sc_gather()(codebook, idx.reshape(N_TOK))
    q = rows.reshape(B, H, W, C)
    return jnp.transpose(q, (0, 3, 1, 2))


# speedup vs baseline: 1.0126x; 1.0126x over previous
import functools

import jax
import jax.numpy as jnp
from jax import lax
from jax.experimental import pallas as pl
from jax.experimental.pallas import tpu as pltpu
from jax.experimental.pallas import tpu_sc as plsc

B, C, H, W = 8, 64, 32, 32
TOK_PER_B = H * W
N_CODES = 8192
CODE_TILE = 8192
N_CT = N_CODES // CODE_TILE
VR_PER_TILE = CODE_TILE // 8

SC_NC, SC_NS = 2, 16
SC_NW = SC_NC * SC_NS
N_TOK = B * TOK_PER_B
TOK_PER_W = N_TOK // SC_NW


SUB = 4
SUB_ROWS = CODE_TILE // SUB


def _subtile_argmax(s):
    t = s.shape[-1]
    nvr = SUB_ROWS // 8
    v3 = s.reshape(nvr // 2, 16, t)
    a, b = v3[:, :8, :], v3[:, 8:, :]
    cmp = a >= b
    val = jnp.where(cmp, a, b)
    idx = jnp.where(cmp, jnp.int32(0), jnp.int32(1))
    groups = nvr // 2
    k = 1
    while groups > 1:
        groups //= 2
        val = val.reshape(groups, 16, t)
        idx = idx.reshape(groups, 16, t)
        a, b = val[:, :8, :], val[:, 8:, :]
        ia, ib = idx[:, :8, :], idx[:, 8:, :]
        cmp = a >= b
        val = jnp.where(cmp, a, b)
        idx = jnp.where(cmp, ia, ib + jnp.int32(1 << k))
        k += 1
    return val.reshape(8, t), idx.reshape(8, t)


BPS = 2


def _argmax_body(x_ref, cb_ref, out_ref):
    for g in range(BPS):
        _one_image(x_ref, cb_ref, out_ref, g)


def _one_image(x_ref, cb_ref, out_ref, g):
    xb = x_ref[g]
    norm = jnp.sqrt(jnp.sum(xb * xb, axis=0, keepdims=True))
    xn = xb / jnp.maximum(norm, 1e-12)

    def sub_dot(i):
        return lax.dot_general(
            cb_ref[pl.ds(i * SUB_ROWS, SUB_ROWS), :], xn,
            (((1,), (0,)), ((), ())),
            preferred_element_type=jnp.float32,
            precision=lax.Precision.DEFAULT,
        )

    results = []
    s_prev = sub_dot(0)
    for i in range(1, SUB + 1):
        s_next = sub_dot(i) if i < SUB else None
        v, ix = _subtile_argmax(s_prev)
        results.append((v, ix * 8 + (i - 1) * SUB_ROWS))
        s_prev = s_next

    val, row = results[0]
    srow = lax.broadcasted_iota(jnp.int32, (8, TOK_PER_B), 0)
    row = row + srow
    for v, r in results[1:]:
        r = r + srow
        cmp = val >= v
        val = jnp.where(cmp, val, v)
        row = jnp.where(cmp, row, r)
    sub = 8
    while sub > 1:
        sub //= 2
        a, b = val[:sub], val[sub:]
        ra, rb = row[:sub], row[sub:]
        win_a = (a > b) | ((a == b) & (ra < rb))
        val = jnp.where(win_a, a, b)
        row = jnp.where(win_a, ra, rb)
    out_ref[g] = row.reshape(1, TOK_PER_B)


def _nearest_code(xr, codebook):
    return pl.pallas_call(
        _argmax_body,
        grid=(B // BPS,),
        in_specs=[
            pl.BlockSpec((BPS, C, TOK_PER_B), lambda b: (b, 0, 0)),
            pl.BlockSpec((CODE_TILE, C), lambda b: (0, 0)),
        ],
        out_specs=pl.BlockSpec((BPS, 1, TOK_PER_B), lambda b: (b, 0, 0)),
        out_shape=jax.ShapeDtypeStruct((B, 1, TOK_PER_B), jnp.int32),
    )(xr, codebook)


@functools.cache
def _make_sc_gather():
    @functools.partial(
        pl.kernel,
        out_type=jax.ShapeDtypeStruct((N_TOK, C), jnp.float32),
        mesh=plsc.VectorSubcoreMesh(core_axis_name="c", subcore_axis_name="s"),
        compiler_params=pltpu.CompilerParams(use_tc_tiling_on_sc=False),
        scratch_types=[
            pltpu.VMEM((TOK_PER_W,), jnp.int32),
            pltpu.VMEM((TOK_PER_W, C), jnp.float32),
            pltpu.SemaphoreType.DMA,
        ],
    )
    def _sc_gather(table_hbm, idx_hbm, out_hbm, idx_v, rows_v, sem):
        wid = lax.axis_index("s") * SC_NC + lax.axis_index("c")
        base = wid * TOK_PER_W
        pltpu.sync_copy(idx_hbm.at[pl.ds(base, TOK_PER_W)], idx_v)
        pltpu.async_copy(table_hbm.at[idx_v], rows_v, sem).wait()
        pltpu.sync_copy(rows_v, out_hbm.at[pl.ds(base, TOK_PER_W)])

    return _sc_gather


def kernel(x, codebook):
    xr = x.reshape(B, C, TOK_PER_B)
    idx = _nearest_code(xr, codebook)
    rows = _make_sc_gather()(codebook, idx.reshape(N_TOK))
    q = rows.reshape(B, H, W, C)
    return jnp.transpose(q, (0, 3, 1, 2))


# speedup vs baseline: 1.0196x; 1.0069x over previous
import functools

import jax
import jax.numpy as jnp
from jax import lax
from jax.experimental import pallas as pl
from jax.experimental.pallas import tpu as pltpu
from jax.experimental.pallas import tpu_sc as plsc

B, C, H, W = 8, 64, 32, 32
TOK_PER_B = H * W
N_CODES = 8192
CODE_TILE = 8192
N_CT = N_CODES // CODE_TILE
VR_PER_TILE = CODE_TILE // 8

SC_NC, SC_NS = 2, 16
SC_NW = SC_NC * SC_NS
N_TOK = B * TOK_PER_B
TOK_PER_W = N_TOK // SC_NW


SUB = 4
SUB_ROWS = CODE_TILE // SUB


def _subtile_argmax(s):
    t = s.shape[-1]
    nvr = SUB_ROWS // 8
    v3 = s.reshape(nvr // 2, 16, t)
    a, b = v3[:, :8, :], v3[:, 8:, :]
    cmp = a >= b
    val = jnp.where(cmp, a, b)
    idx = jnp.where(cmp, jnp.int32(0), jnp.int32(1))
    groups = nvr // 2
    k = 1
    while groups > 1:
        groups //= 2
        val = val.reshape(groups, 16, t)
        idx = idx.reshape(groups, 16, t)
        a, b = val[:, :8, :], val[:, 8:, :]
        ia, ib = idx[:, :8, :], idx[:, 8:, :]
        cmp = a >= b
        val = jnp.where(cmp, a, b)
        idx = jnp.where(cmp, ia, ib + jnp.int32(1 << k))
        k += 1
    return val.reshape(8, t), idx.reshape(8, t)


BPS = 4


def _argmax_body(x_ref, cb_ref, out_ref):
    for g in range(BPS):
        _one_image(x_ref, cb_ref, out_ref, g)


def _one_image(x_ref, cb_ref, out_ref, g):
    xb = x_ref[g]
    norm = jnp.sqrt(jnp.sum(xb * xb, axis=0, keepdims=True))
    xn = xb / jnp.maximum(norm, 1e-12)

    def sub_dot(i):
        return lax.dot_general(
            cb_ref[pl.ds(i * SUB_ROWS, SUB_ROWS), :], xn,
            (((1,), (0,)), ((), ())),
            preferred_element_type=jnp.float32,
            precision=lax.Precision.DEFAULT,
        )

    results = []
    s_prev = sub_dot(0)
    for i in range(1, SUB + 1):
        s_next = sub_dot(i) if i < SUB else None
        v, ix = _subtile_argmax(s_prev)
        results.append((v, ix * 8 + (i - 1) * SUB_ROWS))
        s_prev = s_next

    val, row = results[0]
    srow = lax.broadcasted_iota(jnp.int32, (8, TOK_PER_B), 0)
    row = row + srow
    for v, r in results[1:]:
        r = r + srow
        cmp = val >= v
        val = jnp.where(cmp, val, v)
        row = jnp.where(cmp, row, r)
    sub = 8
    while sub > 1:
        sub //= 2
        a, b = val[:sub], val[sub:]
        ra, rb = row[:sub], row[sub:]
        win_a = (a > b) | ((a == b) & (ra < rb))
        val = jnp.where(win_a, a, b)
        row = jnp.where(win_a, ra, rb)
    out_ref[g] = row.reshape(1, TOK_PER_B)


def _nearest_code(xr, codebook):
    return pl.pallas_call(
        _argmax_body,
        grid=(B // BPS,),
        in_specs=[
            pl.BlockSpec((BPS, C, TOK_PER_B), lambda b: (b, 0, 0)),
            pl.BlockSpec((CODE_TILE, C), lambda b: (0, 0)),
        ],
        out_specs=pl.BlockSpec((BPS, 1, TOK_PER_B), lambda b: (b, 0, 0)),
        out_shape=jax.ShapeDtypeStruct((B, 1, TOK_PER_B), jnp.int32),
    )(xr, codebook)


@functools.cache
def _make_sc_gather():
    @functools.partial(
        pl.kernel,
        out_type=jax.ShapeDtypeStruct((N_TOK, C), jnp.float32),
        mesh=plsc.VectorSubcoreMesh(core_axis_name="c", subcore_axis_name="s"),
        compiler_params=pltpu.CompilerParams(use_tc_tiling_on_sc=False),
        scratch_types=[
            pltpu.VMEM((TOK_PER_W,), jnp.int32),
            pltpu.VMEM((TOK_PER_W, C), jnp.float32),
            pltpu.SemaphoreType.DMA,
        ],
    )
    def _sc_gather(table_hbm, idx_hbm, out_hbm, idx_v, rows_v, sem):
        wid = lax.axis_index("s") * SC_NC + lax.axis_index("c")
        base = wid * TOK_PER_W
        pltpu.sync_copy(idx_hbm.at[pl.ds(base, TOK_PER_W)], idx_v)
        pltpu.async_copy(table_hbm.at[idx_v], rows_v, sem).wait()
        pltpu.sync_copy(rows_v, out_hbm.at[pl.ds(base, TOK_PER_W)])

    return _sc_gather


def kernel(x, codebook):
    xr = x.reshape(B, C, TOK_PER_B)
    idx = _nearest_code(xr, codebook)
    rows = _make_sc_gather()(codebook, idx.reshape(N_TOK))
    q = rows.reshape(B, H, W, C)
    return jnp.transpose(q, (0, 3, 1, 2))
